# baseline (speedup 1.0000x reference)
import functools

import jax
import jax.numpy as jnp
from jax import lax
from jax.experimental import pallas as pl
from jax.experimental.pallas import tpu as pltpu
from jax.experimental.pallas import tpu_sc as plsc

_INTERVAL = [200000, 150000, 150000, 100000, 100000, 100000, 100000, 50000, 50000]
_OFFS = tuple(sum(_INTERVAL[:j]) for j in range(len(_INTERVAL)))
_V = sum(_INTERVAL)

_B = 16384
_D = 32
_K = 9
_NC = 2
_NS = 16
_NW = _NC * _NS
_BPW = _B // _NW
_CHB = 128
_NCH = _BPW // _CHB
_L = 16

_QS = 250880
_NB = 25088
_NST = _QS // _NB
_T3B = 3 * _QS // _NB
_NT3 = (_V - 3 * _QS) // _NB
_TAIL0 = 3 * _QS + _NT3 * _NB
_MAXB = _V // _NB - 1


def _repack_body(t0, t1, t2, t3, t4, o_ref):
    pid = pl.program_id(0)

    @pl.when(pid < _NT3)
    def _():
        x = jnp.concatenate([t0[...], t1[...], t2[...], t3[...]], axis=0)
        o_ref[...] = x.T

    @pl.when(pid >= _NT3)
    def _():
        x = jnp.concatenate([t0[...], t1[...], t2[...], t4[...]], axis=0)
        o_ref[...] = x.T


def _sc_body(packed_hbm, dataT_hbm, attn_hbm, out_hbm,
             d_v, idx_v, rows_v, out_v, attn_v, *sems):
    wid = lax.axis_index("s") * _NC + lax.axis_index("c")
    base = wid * _BPW

    pltpu.sync_copy(dataT_hbm.at[pl.ds(0, _K), pl.ds(base, _BPW)], d_v)
    pltpu.sync_copy(attn_hbm, attn_v)

    iota = lax.iota(jnp.int32, _L)
    av = attn_v[...]
    wgt = [jnp.full((_L,), jnp.sum(jnp.where(iota == j, av, 0.0)), jnp.float32)
           for j in range(_K)]

    def fire(g, par):
        cb = g * _CHB
        for j in range(_K):
            for q in range(_CHB // _L):
                idx = d_v[j, pl.ds(cb + 16 * q, _L)] + _OFFS[j]
                s = ((idx >= _QS).astype(jnp.int32)
                     + (idx >= 2 * _QS).astype(jnp.int32)
                     + (idx >= 3 * _QS).astype(jnp.int32))
                idx_v[par, j, pl.ds(16 * q, _L)] = (
                    lax.shift_left(idx - s * _QS, 2) + s)
        return [pltpu.async_copy(packed_hbm.at[idx_v.at[par, j]],
                                 rows_v.at[par, pl.ds(j * _CHB, _CHB)],
                                 sems[par])
                for j in range(_K)]

    cps = fire(0, 0)
    for g in range(_NCH):
        par = g & 1
        for c in cps:
            c.wait()
        if g + 1 < _NCH:
            cps = fire(g + 1, (g + 1) & 1)

        def b_body(b, c2, _par=par):
            for h in range(_D // _L):
                sl = pl.ds(16 * h, _L)
                acc = rows_v[_par, b, sl] * wgt[0]
                for j in range(1, _K):
                    acc = acc + rows_v[_par, j * _CHB + b, sl] * wgt[j]
                out_v[b, sl] = acc
            return c2
        lax.fori_loop(0, _CHB, b_body, 0)
        pltpu.sync_copy(out_v, out_hbm.at[pl.ds(base + g * _CHB, _CHB)])


@jax.jit
def _emb(tableT, tail2, dataT, attn16):
    packed = pl.pallas_call(
        _repack_body,
        grid=(_NST,),
        in_specs=[
            pl.BlockSpec((_D, _NB), lambda k: (0, k)),
            pl.BlockSpec((_D, _NB), lambda k: (0, _NST + k)),
            pl.BlockSpec((_D, _NB), lambda k: (0, 2 * _NST + k)),
            pl.BlockSpec((_D, _NB),
                         lambda k: (0, jnp.minimum(_T3B + k, _MAXB))),
            pl.BlockSpec((_D, _NB), lambda k: (0, 0)),
        ],
        out_specs=pl.BlockSpec((_NB, 128), lambda k: (k, 0)),
        out_shape=jax.ShapeDtypeStruct((_QS, 128), jnp.float32),
        compiler_params=pltpu.CompilerParams(
            vmem_limit_bytes=120 * 1024 * 1024),
    )(tableT, tableT, tableT, tableT, tail2)
    packed32 = packed.reshape(4 * _QS, _D)

    mesh = plsc.VectorSubcoreMesh(core_axis_name="c", subcore_axis_name="s")
    return pl.kernel(
        _sc_body,
        out_type=jax.ShapeDtypeStruct((_B, _D), jnp.float32),
        mesh=mesh,
        compiler_params=pltpu.CompilerParams(needs_layout_passes=False,
                                             use_tc_tiling_on_sc=False),
        scratch_types=[
            pltpu.VMEM((_K, _BPW), jnp.int32),
            pltpu.VMEM((2, _K, _CHB), jnp.int32),
            pltpu.VMEM((2, _K * _CHB, _D), jnp.float32),
            pltpu.VMEM((_CHB, _D), jnp.float32),
            pltpu.VMEM((_L,), jnp.float32),
            pltpu.SemaphoreType.DMA,
            pltpu.SemaphoreType.DMA,
        ],
    )(packed32, dataT, attn16)


def kernel(data, embedding_table, attn_score):
    tableT = embedding_table.T
    tail2 = jnp.pad(tableT[:, _TAIL0:], ((0, 0), (0, _NB - (_V - _TAIL0))))
    dataT = data.T
    attn16 = jnp.pad(attn_score.reshape(_K), (0, _L - _K))
    result = _emb(tableT, tail2, dataT, attn16)
    return (result, attn_score)


# speedup vs baseline: 1.0013x; 1.0013x over previous
import functools

import jax
import jax.numpy as jnp
from jax import lax
from jax.experimental import pallas as pl
from jax.experimental.pallas import tpu as pltpu
from jax.experimental.pallas import tpu_sc as plsc

_INTERVAL = [200000, 150000, 150000, 100000, 100000, 100000, 100000, 50000, 50000]
_OFFS = tuple(sum(_INTERVAL[:j]) for j in range(len(_INTERVAL)))
_V = sum(_INTERVAL)

_B = 16384
_D = 32
_K = 9
_NC = 2
_NS = 16
_NW = _NC * _NS
_BPW = _B // _NW
_CHB = 128
_NCH = _BPW // _CHB
_L = 16

_QS = 250880
_NB = 17920
_NST = _QS // _NB
_T3B = 3 * _QS // _NB
_NT3 = (_V - 3 * _QS) // _NB
_TAIL0 = 3 * _QS + _NT3 * _NB
_MAXB = _V // _NB - 1


def _repack_body(t0, t1, t2, t3, t4, o_ref):
    pid = pl.program_id(0)

    @pl.when(pid < _NT3)
    def _():
        x = jnp.concatenate([t0[...], t1[...], t2[...], t3[...]], axis=0)
        o_ref[...] = x.T

    @pl.when(pid >= _NT3)
    def _():
        x = jnp.concatenate([t0[...], t1[...], t2[...], t4[...]], axis=0)
        o_ref[...] = x.T


def _sc_body(packed_hbm, dataT_hbm, attn_hbm, out_hbm,
             d_v, idx_v, rows_v, out_v, attn_v, *sems):
    wid = lax.axis_index("s") * _NC + lax.axis_index("c")
    base = wid * _BPW

    pltpu.sync_copy(dataT_hbm.at[pl.ds(0, _K), pl.ds(base, _BPW)], d_v)
    pltpu.sync_copy(attn_hbm, attn_v)

    iota = lax.iota(jnp.int32, _L)
    av = attn_v[...]
    wgt = [jnp.full((_L,), jnp.sum(jnp.where(iota == j, av, 0.0)), jnp.float32)
           for j in range(_K)]

    def fire(g, par):
        cb = g * _CHB
        for j in range(_K):
            for q in range(_CHB // _L):
                idx = d_v[j, pl.ds(cb + 16 * q, _L)] + _OFFS[j]
                s = ((idx >= _QS).astype(jnp.int32)
                     + (idx >= 2 * _QS).astype(jnp.int32)
                     + (idx >= 3 * _QS).astype(jnp.int32))
                idx_v[par, j, pl.ds(16 * q, _L)] = (
                    lax.shift_left(idx - s * _QS, 2) + s)
        return [pltpu.async_copy(packed_hbm.at[idx_v.at[par, j]],
                                 rows_v.at[par, pl.ds(j * _CHB, _CHB)],
                                 sems[par])
                for j in range(_K)]

    cps = fire(0, 0)
    for g in range(_NCH):
        par = g & 1
        for c in cps:
            c.wait()
        if g + 1 < _NCH:
            cps = fire(g + 1, (g + 1) & 1)

        def b_body(b, c2, _par=par):
            for h in range(_D // _L):
                sl = pl.ds(16 * h, _L)
                acc = rows_v[_par, b, sl] * wgt[0]
                for j in range(1, _K):
                    acc = acc + rows_v[_par, j * _CHB + b, sl] * wgt[j]
                out_v[b, sl] = acc
            return c2
        lax.fori_loop(0, _CHB, b_body, 0, unroll=2)
        pltpu.sync_copy(out_v, out_hbm.at[pl.ds(base + g * _CHB, _CHB)])


@jax.jit
def _emb(tableT, tail2, dataT, attn16):
    packed = pl.pallas_call(
        _repack_body,
        grid=(_NST,),
        in_specs=[
            pl.BlockSpec((_D, _NB), lambda k: (0, k)),
            pl.BlockSpec((_D, _NB), lambda k: (0, _NST + k)),
            pl.BlockSpec((_D, _NB), lambda k: (0, 2 * _NST + k)),
            pl.BlockSpec((_D, _NB),
                         lambda k: (0, jnp.minimum(_T3B + k, _MAXB))),
            pl.BlockSpec((_D, _NB), lambda k: (0, 0)),
        ],
        out_specs=pl.BlockSpec((_NB, 128), lambda k: (k, 0)),
        out_shape=jax.ShapeDtypeStruct((_QS, 128), jnp.float32),
        compiler_params=pltpu.CompilerParams(
            vmem_limit_bytes=120 * 1024 * 1024),
    )(tableT, tableT, tableT, tableT, tail2)
    packed32 = packed.reshape(4 * _QS, _D)

    mesh = plsc.VectorSubcoreMesh(core_axis_name="c", subcore_axis_name="s")
    return pl.kernel(
        _sc_body,
        out_type=jax.ShapeDtypeStruct((_B, _D), jnp.float32),
        mesh=mesh,
        compiler_params=pltpu.CompilerParams(needs_layout_passes=False,
                                             use_tc_tiling_on_sc=False),
        scratch_types=[
            pltpu.VMEM((_K, _BPW), jnp.int32),
            pltpu.VMEM((2, _K, _CHB), jnp.int32),
            pltpu.VMEM((2, _K * _CHB, _D), jnp.float32),
            pltpu.VMEM((_CHB, _D), jnp.float32),
            pltpu.VMEM((_L,), jnp.float32),
            pltpu.SemaphoreType.DMA,
            pltpu.SemaphoreType.DMA,
        ],
    )(packed32, dataT, attn16)


def kernel(data, embedding_table, attn_score):
    tableT = embedding_table.T
    tail2 = jnp.pad(tableT[:, _TAIL0:], ((0, 0), (0, _NB - (_V - _TAIL0))))
    dataT = data.T
    attn16 = jnp.pad(attn_score.reshape(_K), (0, _L - _K))
    result = _emb(tableT, tail2, dataT, attn16)
    return (result, attn_score)


# speedup vs baseline: 1.0356x; 1.0343x over previous
import functools

import jax
import jax.numpy as jnp
from jax import lax
from jax.experimental import pallas as pl
from jax.experimental.pallas import tpu as pltpu
from jax.experimental.pallas import tpu_sc as plsc

_INTERVAL = [200000, 150000, 150000, 100000, 100000, 100000, 100000, 50000, 50000]
_OFFS = tuple(sum(_INTERVAL[:j]) for j in range(len(_INTERVAL)))
_V = sum(_INTERVAL)

_B = 16384
_D = 32
_K = 9
_NC = 2
_NS = 16
_NW = _NC * _NS
_BPW = _B // _NW
_CHB = 128
_NCH = _BPW // _CHB
_L = 16

_QS = 250880
_NB = 17920
_NST = _QS // _NB
_T3B = 3 * _QS // _NB
_NT3 = (_V - 3 * _QS) // _NB
_TAIL0 = 3 * _QS + _NT3 * _NB
_MAXB = _V // _NB - 1


def _repack_body(t0, t1, t2, t3, t4, o_ref):
    pid = pl.program_id(0)

    @pl.when(pid < _NT3)
    def _():
        x = jnp.concatenate([t0[...], t1[...], t2[...], t3[...]], axis=0)
        o_ref[...] = x.T

    @pl.when(pid >= _NT3)
    def _():
        x = jnp.concatenate([t0[...], t1[...], t2[...], t4[...]], axis=0)
        o_ref[...] = x.T


def _sc_body(packed_hbm, dataT_hbm, attn_hbm, out_hbm,
             d_v, idx_v, rows_v, out_v, attn_v, *sems):
    wid = lax.axis_index("s") * _NC + lax.axis_index("c")
    base = wid * _BPW

    pltpu.sync_copy(dataT_hbm.at[pl.ds(0, _K), pl.ds(base, _BPW)], d_v)
    pltpu.sync_copy(attn_hbm, attn_v)

    iota = lax.iota(jnp.int32, _L)
    zero = jnp.zeros((_L,), jnp.int32)
    av = attn_v[...]
    wgt = [jnp.full((_L,), jnp.sum(jnp.where(iota == j, av, 0.0)), jnp.float32)
           for j in range(_K)]

    def fire(g, par):
        cb = g * _CHB
        for j in range(_K):
            for q in range(_CHB // _L):
                idx = d_v[j, pl.ds(cb + 16 * q, _L)] + _OFFS[j]
                s = ((idx >= _QS).astype(jnp.int32)
                     + (idx >= 2 * _QS).astype(jnp.int32)
                     + (idx >= 3 * _QS).astype(jnp.int32))
                idx_v[par, j, pl.ds(16 * q, _L)] = (
                    lax.shift_left(idx - s * _QS, 2) + s)
        return [pltpu.async_copy(packed_hbm.at[idx_v.at[par, j]],
                                 rows_v.at[par, pl.ds(j * _CHB, _CHB)],
                                 sems[par])
                for j in range(_K)]

    cps = fire(0, 0)
    for g in range(_NCH):
        par = g & 1
        for c in cps:
            c.wait()
        if g + 1 < _NCH:
            cps = fire(g + 1, (g + 1) & 1)

        def b_body(b, c2, _par=par):
            bs = zero + b
            for h in range(_D // _L):
                sl = pl.ds(16 * h, _L)
                acc = rows_v[_par, b, sl] * wgt[0]
                for j in range(1, _K):
                    acc = acc + rows_v[_par, j * _CHB + b, sl] * wgt[j]
                plsc.store_scatter(out_v, [iota + 16 * h, bs], acc)
            return c2
        lax.fori_loop(0, _CHB, b_body, 0)
        ct = 4 * wid + g
        for r in range(_D // 8):
            pltpu.sync_copy(out_v.at[pl.ds(8 * r, 8), pl.ds(0, _CHB)],
                            out_hbm.at[pl.ds(1024 * r + 8 * ct, 8)])


@jax.jit
def _emb(tableT, tail2, dataT, attn16):
    packed = pl.pallas_call(
        _repack_body,
        grid=(_NST,),
        in_specs=[
            pl.BlockSpec((_D, _NB), lambda k: (0, k)),
            pl.BlockSpec((_D, _NB), lambda k: (0, _NST + k)),
            pl.BlockSpec((_D, _NB), lambda k: (0, 2 * _NST + k)),
            pl.BlockSpec((_D, _NB),
                         lambda k: (0, jnp.minimum(_T3B + k, _MAXB))),
            pl.BlockSpec((_D, _NB), lambda k: (0, 0)),
        ],
        out_specs=pl.BlockSpec((_NB, 128), lambda k: (k, 0)),
        out_shape=jax.ShapeDtypeStruct((_QS, 128), jnp.float32),
        compiler_params=pltpu.CompilerParams(
            vmem_limit_bytes=120 * 1024 * 1024),
    )(tableT, tableT, tableT, tableT, tail2)
    packed32 = packed.reshape(4 * _QS, _D)

    mesh = plsc.VectorSubcoreMesh(core_axis_name="c", subcore_axis_name="s")
    return pl.kernel(
        _sc_body,
        out_type=jax.ShapeDtypeStruct((_B // 4, 128), jnp.float32),
        mesh=mesh,
        compiler_params=pltpu.CompilerParams(needs_layout_passes=False,
                                             use_tc_tiling_on_sc=False),
        scratch_types=[
            pltpu.VMEM((_K, _BPW), jnp.int32),
            pltpu.VMEM((2, _K, _CHB), jnp.int32),
            pltpu.VMEM((2, _K * _CHB, _D), jnp.float32),
            pltpu.VMEM((_D, _CHB + 1), jnp.float32),
            pltpu.VMEM((_L,), jnp.float32),
            pltpu.SemaphoreType.DMA,
            pltpu.SemaphoreType.DMA,
        ],
    )(packed32, dataT, attn16)


def kernel(data, embedding_table, attn_score):
    tableT = embedding_table.T
    tail2 = jnp.pad(tableT[:, _TAIL0:], ((0, 0), (0, _NB - (_V - _TAIL0))))
    dataT = data.T
    attn16 = jnp.pad(attn_score.reshape(_K), (0, _L - _K))
    out4096 = _emb(tableT, tail2, dataT, attn16)
    result = (out4096.reshape(4, 128, 8, 128)
              .transpose(0, 2, 1, 3)
              .reshape(_D, _B).T)
    return (result, attn_score)


# speedup vs baseline: 1.0376x; 1.0019x over previous
import functools

import jax
import jax.numpy as jnp
from jax import lax
from jax.experimental import pallas as pl
from jax.experimental.pallas import tpu as pltpu
from jax.experimental.pallas import tpu_sc as plsc

_INTERVAL = [200000, 150000, 150000, 100000, 100000, 100000, 100000, 50000, 50000]
_OFFS = tuple(sum(_INTERVAL[:j]) for j in range(len(_INTERVAL)))
_V = sum(_INTERVAL)

_B = 16384
_D = 32
_K = 9
_NC = 2
_NS = 16
_NW = _NC * _NS
_BPW = _B // _NW
_CHB = 128
_NCH = _BPW // _CHB
_L = 16

_QS = 250880
_NB = 17920
_NST = _QS // _NB
_T3B = 3 * _QS // _NB
_NT3 = (_V - 3 * _QS) // _NB
_TAIL0 = 3 * _QS + _NT3 * _NB
_MAXB = _V // _NB - 1


def _repack_body(t0, t1, t2, t3, t4, o_ref):
    pid = pl.program_id(0)

    @pl.when(pid < _NT3)
    def _():
        x = jnp.concatenate([t0[...], t1[...], t2[...], t3[...]], axis=0)
        o_ref[...] = x.T

    @pl.when(pid >= _NT3)
    def _():
        x = jnp.concatenate([t0[...], t1[...], t2[...], t4[...]], axis=0)
        o_ref[...] = x.T


def _sc_body(packed_hbm, dataT_hbm, attn_hbm, out_hbm,
             d_v, idx_v, rows_v, out_v, attn_v, *sems):
    wid = lax.axis_index("s") * _NC + lax.axis_index("c")
    base = wid * _BPW

    pltpu.sync_copy(dataT_hbm.at[pl.ds(0, _K), pl.ds(base, _BPW)], d_v)
    pltpu.sync_copy(attn_hbm, attn_v)

    iota = lax.iota(jnp.int32, _L)
    zero = jnp.zeros((_L,), jnp.int32)
    av = attn_v[...]
    wgt = [jnp.full((_L,), jnp.sum(jnp.where(iota == j, av, 0.0)), jnp.float32)
           for j in range(_K)]

    def fire(g, par):
        cb = g * _CHB
        for j in range(_K):
            for q in range(_CHB // _L):
                idx = d_v[j, pl.ds(cb + 16 * q, _L)] + _OFFS[j]
                s = ((idx >= _QS).astype(jnp.int32)
                     + (idx >= 2 * _QS).astype(jnp.int32)
                     + (idx >= 3 * _QS).astype(jnp.int32))
                idx_v[par, pl.ds(j * _CHB + 16 * q, _L)] = (
                    lax.shift_left(idx - s * _QS, 2) + s)
        return [pltpu.async_copy(packed_hbm.at[idx_v.at[par]],
                                 rows_v.at[par], sems[par])]

    cps = fire(0, 0)
    for g in range(_NCH):
        par = g & 1
        for c in cps:
            c.wait()
        if g + 1 < _NCH:
            cps = fire(g + 1, (g + 1) & 1)

        def b_body(b, c2, _par=par):
            bs = zero + b
            for h in range(_D // _L):
                sl = pl.ds(16 * h, _L)
                acc = rows_v[_par, b, sl] * wgt[0]
                for j in range(1, _K):
                    acc = acc + rows_v[_par, j * _CHB + b, sl] * wgt[j]
                plsc.store_scatter(out_v, [iota + 16 * h, bs], acc)
            return c2
        lax.fori_loop(0, _CHB, b_body, 0)
        ct = 4 * wid + g
        for r in range(_D // 8):
            pltpu.sync_copy(out_v.at[pl.ds(8 * r, 8), pl.ds(0, _CHB)],
                            out_hbm.at[pl.ds(1024 * r + 8 * ct, 8)])


@jax.jit
def _emb(tableT, tail2, dataT, attn16):
    packed = pl.pallas_call(
        _repack_body,
        grid=(_NST,),
        in_specs=[
            pl.BlockSpec((_D, _NB), lambda k: (0, k)),
            pl.BlockSpec((_D, _NB), lambda k: (0, _NST + k)),
            pl.BlockSpec((_D, _NB), lambda k: (0, 2 * _NST + k)),
            pl.BlockSpec((_D, _NB),
                         lambda k: (0, jnp.minimum(_T3B + k, _MAXB))),
            pl.BlockSpec((_D, _NB), lambda k: (0, 0)),
        ],
        out_specs=pl.BlockSpec((_NB, 128), lambda k: (k, 0)),
        out_shape=jax.ShapeDtypeStruct((_QS, 128), jnp.float32),
        compiler_params=pltpu.CompilerParams(
            vmem_limit_bytes=120 * 1024 * 1024),
    )(tableT, tableT, tableT, tableT, tail2)
    packed32 = packed.reshape(4 * _QS, _D)

    mesh = plsc.VectorSubcoreMesh(core_axis_name="c", subcore_axis_name="s")
    return pl.kernel(
        _sc_body,
        out_type=jax.ShapeDtypeStruct((_B // 4, 128), jnp.float32),
        mesh=mesh,
        compiler_params=pltpu.CompilerParams(needs_layout_passes=False,
                                             use_tc_tiling_on_sc=False),
        scratch_types=[
            pltpu.VMEM((_K, _BPW), jnp.int32),
            pltpu.VMEM((2, _K * _CHB), jnp.int32),
            pltpu.VMEM((2, _K * _CHB, _D), jnp.float32),
            pltpu.VMEM((_D, _CHB + 1), jnp.float32),
            pltpu.VMEM((_L,), jnp.float32),
            pltpu.SemaphoreType.DMA,
            pltpu.SemaphoreType.DMA,
        ],
    )(packed32, dataT, attn16)


def kernel(data, embedding_table, attn_score):
    tableT = embedding_table.T
    tail2 = jnp.pad(tableT[:, _TAIL0:], ((0, 0), (0, _NB - (_V - _TAIL0))))
    dataT = data.T
    attn16 = jnp.pad(attn_score.reshape(_K), (0, _L - _K))
    out4096 = _emb(tableT, tail2, dataT, attn16)
    result = (out4096.reshape(4, 128, 8, 128)
              .transpose(0, 2, 1, 3)
              .reshape(_D, _B).T)
    return (result, attn_score)
